# baseline (speedup 1.0000x reference)
import functools

import jax
import jax.numpy as jnp
from jax import lax
from jax.experimental import pallas as pl
from jax.experimental.pallas import tpu as pltpu
from jax.experimental.pallas import tpu_sc as plsc

NC, NS = 2, 16
NW = NC * NS
NBUF = 4
LANES = 128
NSPLIT = 2


def _emb_call(Bc, boff, H, D, table, xp):
    mesh = plsc.VectorSubcoreMesh(core_axis_name="c", subcore_axis_name="s")
    b_per_w = Bc // NW
    n_super = b_per_w // NBUF

    @functools.partial(
        pl.kernel,
        out_type=jax.ShapeDtypeStruct((Bc, H, D), jnp.float32),
        mesh=mesh,
        scratch_types=[
            pltpu.VMEM((b_per_w, LANES), jnp.int32),
            pltpu.VMEM((NBUF, H, D), jnp.float32),
        ]
        + [pltpu.SemaphoreType.DMA] * (2 * NBUF),
    )
    def emb(table_hbm, idx_hbm, out_hbm, idx_v, rows_v, *sems):
        g_sems, w_sems = sems[:NBUF], sems[NBUF:]
        wid = lax.axis_index("s") * NC + lax.axis_index("c")
        bbase = wid * b_per_w
        pltpu.sync_copy(idx_hbm.at[pl.ds(boff + bbase, b_per_w)], idx_v)

        def super_body(g, carry):
            gathers = []
            for b in range(NBUF):
                @pl.when(g > 0)
                def _():
                    pltpu.make_async_copy(
                        rows_v.at[b], out_hbm.at[bbase], w_sems[b]
                    ).wait()

                r = g * NBUF + b
                gathers.append(
                    pltpu.async_copy(
                        table_hbm.at[idx_v.at[r, pl.ds(0, H)]],
                        rows_v.at[b],
                        g_sems[b],
                    )
                )
            for b in range(NBUF):
                r = g * NBUF + b
                gathers[b].wait()
                pltpu.async_copy(
                    rows_v.at[b], out_hbm.at[bbase + r], w_sems[b]
                )
            return carry

        lax.fori_loop(0, n_super, super_body, 0)
        for b in range(NBUF):
            pltpu.make_async_copy(
                rows_v.at[b], out_hbm.at[bbase], w_sems[b]
            ).wait()

    return emb(table, xp)


def kernel(x, embed_weight):
    B, H = x.shape
    V, D = embed_weight.shape
    xp = jnp.pad(x.astype(jnp.int32), ((0, 0), (0, LANES - H)))
    Bc = B // NSPLIT
    parts = [
        _emb_call(Bc, s * Bc, H, D, embed_weight, xp) for s in range(NSPLIT)
    ]
    return jnp.concatenate(parts, axis=0)


# speedup vs baseline: 2.8805x; 2.8805x over previous
import functools

import jax
import jax.numpy as jnp
from jax import lax
from jax.experimental import pallas as pl
from jax.experimental.pallas import tpu as pltpu
from jax.experimental.pallas import tpu_sc as plsc

NC, NS = 2, 16
NW = NC * NS
NBUF = 5


def _emb_call(B, H, D, table, idxt):
    mesh = plsc.VectorSubcoreMesh(core_axis_name="c", subcore_axis_name="s")
    b_per_w = B // NW
    n_super = H // NBUF

    @functools.partial(
        pl.kernel,
        out_type=jax.ShapeDtypeStruct((H, B, D), jnp.float32),
        mesh=mesh,
        scratch_types=[
            pltpu.VMEM((H, b_per_w), jnp.int32),
            pltpu.VMEM((NBUF, b_per_w, D), jnp.float32),
        ]
        + [pltpu.SemaphoreType.DMA] * (2 * NBUF),
    )
    def emb(table_hbm, idx_hbm, out_hbm, idx_v, rows_v, *sems):
        g_sems, w_sems = sems[:NBUF], sems[NBUF:]
        wid = lax.axis_index("s") * NC + lax.axis_index("c")
        bbase = wid * b_per_w
        pltpu.sync_copy(idx_hbm.at[wid], idx_v)

        def super_body(g, carry):
            gathers = []
            for b in range(NBUF):
                @pl.when(g > 0)
                def _():
                    pltpu.make_async_copy(
                        rows_v.at[b],
                        out_hbm.at[0, pl.ds(bbase, b_per_w)],
                        w_sems[b],
                    ).wait()

                h = g * NBUF + b
                gathers.append(
                    pltpu.async_copy(
                        table_hbm.at[idx_v.at[h]], rows_v.at[b], g_sems[b]
                    )
                )
            for b in range(NBUF):
                h = g * NBUF + b
                gathers[b].wait()
                pltpu.async_copy(
                    rows_v.at[b],
                    out_hbm.at[h, pl.ds(bbase, b_per_w)],
                    w_sems[b],
                )
            return carry

        lax.fori_loop(0, n_super, super_body, 0)
        for b in range(NBUF):
            pltpu.make_async_copy(
                rows_v.at[b], out_hbm.at[0, pl.ds(bbase, b_per_w)], w_sems[b]
            ).wait()

    return emb(table, idxt)


def kernel(x, embed_weight):
    B, H = x.shape
    V, D = embed_weight.shape
    b_per_w = B // NW
    idxt = x.astype(jnp.int32).T.reshape(H, NW, b_per_w).transpose(1, 0, 2)
    y = _emb_call(B, H, D, embed_weight, idxt)
    return jnp.transpose(y, (1, 0, 2))


# speedup vs baseline: 2.9528x; 1.0251x over previous
import functools

import jax
import jax.numpy as jnp
from jax import lax
from jax.experimental import pallas as pl
from jax.experimental.pallas import tpu as pltpu
from jax.experimental.pallas import tpu_sc as plsc

NC, NS = 2, 16
NW = NC * NS
SPLITH = 2
NBUF = 10


def _emb_call(B, H, D, table, idxt):
    mesh = plsc.VectorSubcoreMesh(core_axis_name="c", subcore_axis_name="s")
    b_per_w = B // NW
    csz = b_per_w // SPLITH
    n_super = H * SPLITH // NBUF

    @functools.partial(
        pl.kernel,
        out_type=jax.ShapeDtypeStruct((H, B, D), jnp.float32),
        mesh=mesh,
        scratch_types=[
            pltpu.VMEM((H, b_per_w), jnp.int32),
            pltpu.VMEM((NBUF, csz, D), jnp.float32),
        ]
        + [pltpu.SemaphoreType.DMA] * (2 * NBUF),
    )
    def emb(table_hbm, idx_hbm, out_hbm, idx_v, rows_v, *sems):
        g_sems, w_sems = sems[:NBUF], sems[NBUF:]
        wid = lax.axis_index("s") * NC + lax.axis_index("c")
        bbase = wid * b_per_w
        pltpu.sync_copy(idx_hbm.at[wid], idx_v)

        def super_body(g, carry):
            gathers = []
            for b in range(NBUF):
                @pl.when(g > 0)
                def _():
                    pltpu.make_async_copy(
                        rows_v.at[b],
                        out_hbm.at[0, pl.ds(bbase, csz)],
                        w_sems[b],
                    ).wait()

                h = g * (NBUF // SPLITH) + b // SPLITH
                off = (b % SPLITH) * csz
                gathers.append(
                    pltpu.async_copy(
                        table_hbm.at[idx_v.at[h, pl.ds(off, csz)]],
                        rows_v.at[b],
                        g_sems[b],
                    )
                )
            for b in range(NBUF):
                h = g * (NBUF // SPLITH) + b // SPLITH
                off = (b % SPLITH) * csz
                gathers[b].wait()
                pltpu.async_copy(
                    rows_v.at[b],
                    out_hbm.at[h, pl.ds(bbase + off, csz)],
                    w_sems[b],
                )
            return carry

        lax.fori_loop(0, n_super, super_body, 0)
        for b in range(NBUF):
            pltpu.make_async_copy(
                rows_v.at[b], out_hbm.at[0, pl.ds(bbase, csz)], w_sems[b]
            ).wait()

    return emb(table, idxt)


def kernel(x, embed_weight):
    B, H = x.shape
    V, D = embed_weight.shape
    b_per_w = B // NW
    idxt = x.astype(jnp.int32).T.reshape(H, NW, b_per_w).transpose(1, 0, 2)
    y = _emb_call(B, H, D, embed_weight, idxt)
    return jnp.transpose(y, (1, 0, 2))
